# Initial kernel scaffold; baseline (speedup 1.0000x reference)
import jax
import jax.numpy as jnp
from jax.experimental import pallas as pl


def kernel(voxel_features, batch_idx, y_idx, x_idx):
    raise NotImplementedError("write your pallas kernel here")


# speedup vs baseline: 2.3568x; 2.3568x over previous
import functools

import jax
import jax.numpy as jnp
from jax import lax
from jax.experimental import pallas as pl
from jax.experimental.pallas import tpu as pltpu
from jax.experimental.pallas import tpu_sc as plsc

B, C, NY, NX = 4, 64, 496, 432
N_PILLARS = 48000
PLANE = NY * NX
TOT = B * PLANE
LANES = 16

NC, NS = 2, 16
NW = NC * NS
POS_PER_W = TOT // NW

TW = 48128
SENT = N_PILLARS

PCHUNK = 6000
K1_CHUNKS = N_PILLARS // PCHUNK
K1_GROUPS = PCHUNK // LANES

CROWS = 16
CP = CROWS * NX
K2_CHUNKS = NY // CROWS
K2_GROUPS = CP // LANES

_mesh = plsc.VectorSubcoreMesh(core_axis_name="c", subcore_axis_name="s")
_sc_params = pltpu.CompilerParams(needs_layout_passes=False)


def _wid():
    return lax.axis_index("s") * NC + lax.axis_index("c")


def _tr_body(vf_ref, out_ref):
    i = pl.program_id(0)

    @pl.when(i < N_PILLARS // 128)
    def _():
        out_ref[...] = vf_ref[...].T

    @pl.when(i >= N_PILLARS // 128)
    def _():
        out_ref[...] = jnp.zeros((C, 128), jnp.float32)


def _transpose_table(vf):
    return pl.pallas_call(
        _tr_body,
        grid=(TW // 128,),
        in_specs=[pl.BlockSpec((128, C), lambda i: (jnp.minimum(i, N_PILLARS // 128 - 1), 0))],
        out_specs=pl.BlockSpec((C, 128), lambda i: (0, i)),
        out_shape=jax.ShapeDtypeStruct((C, TW), jnp.float32),
    )(vf)


@functools.partial(
    pl.kernel,
    out_type=jax.ShapeDtypeStruct((TOT,), jnp.int32),
    mesh=_mesh,
    compiler_params=_sc_params,
    scratch_types=[
        pltpu.VMEM((POS_PER_W,), jnp.int32),
        pltpu.VMEM((PCHUNK,), jnp.int32),
        pltpu.VMEM((PCHUNK,), jnp.int32),
        pltpu.VMEM((PCHUNK,), jnp.int32),
    ],
)
def _build_pid(b_hbm, y_hbm, x_hbm, pid_hbm, slab, bv, yv, xv):
    wid = _wid()
    lo = wid * POS_PER_W
    lane = lax.iota(jnp.int32, LANES)
    sent_vec = SENT + lane

    def ibody(i, carry):
        slab[pl.ds(i * LANES, LANES)] = sent_vec
        return carry

    lax.fori_loop(0, POS_PER_W // LANES, ibody, 0, unroll=8)

    for ci in range(K1_CHUNKS):
        base = ci * PCHUNK
        pltpu.sync_copy(b_hbm.at[pl.ds(base, PCHUNK)], bv)
        pltpu.sync_copy(y_hbm.at[pl.ds(base, PCHUNK)], yv)
        pltpu.sync_copy(x_hbm.at[pl.ds(base, PCHUNK)], xv)

        def gbody(g, carry):
            off = g * LANES
            bb = bv[pl.ds(off, LANES)]
            yy = yv[pl.ds(off, LANES)]
            xx = xv[pl.ds(off, LANES)]
            flat = bb * PLANE + yy * NX + xx
            loc = flat - lo
            mask = (loc >= 0) & (loc < POS_PER_W)
            safe = jnp.where(mask, loc, 0)
            pidv = (base + off) + lane
            plsc.store_scatter(slab, [safe], pidv, mask=mask)
            return carry

        lax.fori_loop(0, K1_GROUPS, gbody, 0, unroll=4)

    pltpu.sync_copy(slab, pid_hbm.at[pl.ds(lo, POS_PER_W)])


@functools.partial(
    pl.kernel,
    out_type=jax.ShapeDtypeStruct((B * C * PLANE,), jnp.float32),
    mesh=_mesh,
    compiler_params=_sc_params,
    scratch_types=[
        pltpu.VMEM((TW,), jnp.float32),
        pltpu.VMEM((TW,), jnp.float32),
        pltpu.VMEM((CP,), jnp.int32),
        pltpu.VMEM((CP,), jnp.float32),
        pltpu.VMEM((CP,), jnp.float32),
    ],
)
def _expand(pid_hbm, tab_hbm, out_hbm, t0, t1, pidb, ob0, ob1):
    wid = _wid()
    c0 = wid * 2
    pltpu.sync_copy(tab_hbm.at[c0], t0)
    pltpu.sync_copy(tab_hbm.at[c0 + 1], t1)

    for b in range(B):
        def cbody(k, carry):
            src = b * PLANE + k * CP
            pltpu.sync_copy(pid_hbm.at[pl.ds(src, CP)], pidb)

            def gbody(g, carry):
                off = g * LANES
                idx = pidb[pl.ds(off, LANES)]
                ob0[pl.ds(off, LANES)] = plsc.load_gather(t0, [idx])
                ob1[pl.ds(off, LANES)] = plsc.load_gather(t1, [idx])
                return carry

            lax.fori_loop(0, K2_GROUPS, gbody, 0, unroll=8)

            dst0 = (b * C + c0) * PLANE + k * CP
            dst1 = dst0 + PLANE
            pltpu.sync_copy(ob0, out_hbm.at[pl.ds(dst0, CP)])
            pltpu.sync_copy(ob1, out_hbm.at[pl.ds(dst1, CP)])
            return carry

        lax.fori_loop(0, K2_CHUNKS, cbody, 0)


def kernel(voxel_features, batch_idx, y_idx, x_idx):
    vf = voxel_features.astype(jnp.float32)
    bi = batch_idx.astype(jnp.int32)
    yi = y_idx.astype(jnp.int32)
    xi = x_idx.astype(jnp.int32)

    table = _transpose_table(vf)
    pid = _build_pid(bi, yi, xi)
    flat = _expand(pid, table)
    return flat.reshape(B, C, NY, NX)


# speedup vs baseline: 5.1086x; 2.1676x over previous
import functools

import jax
import jax.numpy as jnp
from jax import lax
from jax.experimental import pallas as pl
from jax.experimental.pallas import tpu as pltpu
from jax.experimental.pallas import tpu_sc as plsc

B, C, NY, NX = 4, 64, 496, 432
N_PILLARS = 48000
PLANE = NY * NX
TOT = B * PLANE
LANES = 16

NC, NS = 2, 16
NW = NC * NS
POS_PER_W = TOT // NW

TW = 48128
SENT = N_PILLARS

PCHUNK = 6000
K1_CHUNKS = N_PILLARS // PCHUNK
K1_GROUPS = PCHUNK // LANES

CROWS = 16
CP = CROWS * NY
K2_CHUNKS = NX // CROWS
K2_GROUPS = NY // LANES

_mesh = plsc.VectorSubcoreMesh(core_axis_name="c", subcore_axis_name="s")
_sc_params = pltpu.CompilerParams(needs_layout_passes=False)


def _wid():
    return lax.axis_index("s") * NC + lax.axis_index("c")


def _tr_body(vf_ref, out_ref):
    i = pl.program_id(0)

    @pl.when(i < N_PILLARS // 128)
    def _():
        out_ref[...] = vf_ref[...].T

    @pl.when(i >= N_PILLARS // 128)
    def _():
        out_ref[...] = jnp.zeros((C, 128), jnp.float32)


def _transpose_table(vf):
    return pl.pallas_call(
        _tr_body,
        grid=(TW // 128,),
        in_specs=[pl.BlockSpec((128, C), lambda i: (jnp.minimum(i, N_PILLARS // 128 - 1), 0))],
        out_specs=pl.BlockSpec((C, 128), lambda i: (0, i)),
        out_shape=jax.ShapeDtypeStruct((C, TW), jnp.float32),
    )(vf)


@functools.partial(
    pl.kernel,
    out_type=jax.ShapeDtypeStruct((TOT,), jnp.int32),
    mesh=_mesh,
    compiler_params=_sc_params,
    scratch_types=[
        pltpu.VMEM((POS_PER_W,), jnp.int32),
        pltpu.VMEM((PCHUNK,), jnp.int32),
        pltpu.VMEM((PCHUNK,), jnp.int32),
        pltpu.VMEM((PCHUNK,), jnp.int32),
    ],
)
def _build_pid(b_hbm, y_hbm, x_hbm, pid_hbm, slab, bv, yv, xv):
    wid = _wid()
    lo = wid * POS_PER_W
    lane = lax.iota(jnp.int32, LANES)
    sent_vec = SENT + lane

    def ibody(i, carry):
        slab[pl.ds(i * LANES, LANES)] = sent_vec
        return carry

    lax.fori_loop(0, POS_PER_W // LANES, ibody, 0, unroll=8)

    for ci in range(K1_CHUNKS):
        base = ci * PCHUNK
        pltpu.sync_copy(b_hbm.at[pl.ds(base, PCHUNK)], bv)
        pltpu.sync_copy(y_hbm.at[pl.ds(base, PCHUNK)], yv)
        pltpu.sync_copy(x_hbm.at[pl.ds(base, PCHUNK)], xv)

        def gbody(g, carry):
            off = g * LANES
            bb = bv[pl.ds(off, LANES)]
            yy = yv[pl.ds(off, LANES)]
            xx = xv[pl.ds(off, LANES)]
            flat = bb * PLANE + xx * NY + yy
            loc = flat - lo
            mask = (loc >= 0) & (loc < POS_PER_W)
            safe = jnp.where(mask, loc, 0)
            pidv = (base + off) + lane
            plsc.store_scatter(slab, [safe], pidv, mask=mask)
            return carry

        lax.fori_loop(0, K1_GROUPS, gbody, 0, unroll=4)

    pltpu.sync_copy(slab, pid_hbm.at[pl.ds(lo, POS_PER_W)])


@functools.partial(
    pl.kernel,
    out_type=jax.ShapeDtypeStruct((B, C, NX, NY), jnp.float32),
    mesh=_mesh,
    compiler_params=_sc_params,
    scratch_types=[
        pltpu.VMEM((TW,), jnp.float32),
        pltpu.VMEM((TW,), jnp.float32),
        pltpu.VMEM((CP,), jnp.int32),
        pltpu.VMEM((CROWS, NY), jnp.float32),
        pltpu.VMEM((CROWS, NY), jnp.float32),
    ],
)
def _expand(pid_hbm, tab_hbm, out_hbm, t0, t1, pidb, ob0, ob1):
    wid = _wid()
    c0 = wid * 2
    pltpu.sync_copy(tab_hbm.at[c0], t0)
    pltpu.sync_copy(tab_hbm.at[c0 + 1], t1)

    for b in range(B):
        def cbody(k, carry):
            src = b * PLANE + k * CP
            pltpu.sync_copy(pid_hbm.at[pl.ds(src, CP)], pidb)

            def rbody(r, carry):
                def gbody(cg, carry):
                    off = cg * LANES
                    idx = pidb[pl.ds(r * NY + off, LANES)]
                    ob0[r, pl.ds(off, LANES)] = plsc.load_gather(t0, [idx])
                    ob1[r, pl.ds(off, LANES)] = plsc.load_gather(t1, [idx])
                    return carry

                return lax.fori_loop(0, K2_GROUPS, gbody, carry, unroll=31)

            lax.fori_loop(0, CROWS, rbody, 0)

            pltpu.sync_copy(ob0, out_hbm.at[b, c0, pl.ds(k * CROWS, CROWS)])
            pltpu.sync_copy(ob1, out_hbm.at[b, c0 + 1, pl.ds(k * CROWS, CROWS)])
            return carry

        lax.fori_loop(0, K2_CHUNKS, cbody, 0)


def kernel(voxel_features, batch_idx, y_idx, x_idx):
    vf = voxel_features.astype(jnp.float32)
    bi = batch_idx.astype(jnp.int32)
    yi = y_idx.astype(jnp.int32)
    xi = x_idx.astype(jnp.int32)

    table = _transpose_table(vf)
    pid = _build_pid(bi, yi, xi)
    return jnp.swapaxes(_expand(pid, table), 2, 3)


# speedup vs baseline: 8.9326x; 1.7485x over previous
import functools

import jax
import jax.numpy as jnp
from jax import lax
from jax.experimental import pallas as pl
from jax.experimental.pallas import tpu as pltpu
from jax.experimental.pallas import tpu_sc as plsc

B, C, NY, NX = 4, 64, 496, 432
N_PILLARS = 48000
PLANE = NY * NX
TOT = B * PLANE
LANES = 16

NC, NS = 2, 16
NW = NC * NS
POS_PER_W = TOT // NW

TW = 48128
SENT = N_PILLARS

PCHUNK = 6000
K1_CHUNKS = N_PILLARS // PCHUNK
K1_GROUPS = PCHUNK // LANES

CROWS = 16
CP = CROWS * NY
K2_CHUNKS = NX // CROWS
K2_GROUPS = NY // LANES

_mesh = plsc.VectorSubcoreMesh(core_axis_name="c", subcore_axis_name="s")
_sc_params = pltpu.CompilerParams(needs_layout_passes=False)


def _wid():
    return lax.axis_index("s") * NC + lax.axis_index("c")


def _tr_body(vf_ref, out_ref):
    i = pl.program_id(0)

    @pl.when(i < N_PILLARS // 128)
    def _():
        out_ref[...] = vf_ref[...].T

    @pl.when(i >= N_PILLARS // 128)
    def _():
        out_ref[...] = jnp.zeros((C, 128), jnp.float32)


def _transpose_table(vf):
    return pl.pallas_call(
        _tr_body,
        grid=(TW // 128,),
        in_specs=[pl.BlockSpec((128, C), lambda i: (jnp.minimum(i, N_PILLARS // 128 - 1), 0))],
        out_specs=pl.BlockSpec((C, 128), lambda i: (0, i)),
        out_shape=jax.ShapeDtypeStruct((C, TW), jnp.float32),
    )(vf)


@functools.partial(
    pl.kernel,
    out_type=jax.ShapeDtypeStruct((TOT,), jnp.int32),
    mesh=_mesh,
    compiler_params=_sc_params,
    scratch_types=[
        pltpu.VMEM((POS_PER_W,), jnp.int32),
        pltpu.VMEM((PCHUNK,), jnp.int32),
        pltpu.VMEM((PCHUNK,), jnp.int32),
        pltpu.VMEM((PCHUNK,), jnp.int32),
    ],
)
def _build_pid(b_hbm, y_hbm, x_hbm, pid_hbm, slab, bv, yv, xv):
    wid = _wid()
    lo = wid * POS_PER_W
    lane = lax.iota(jnp.int32, LANES)
    sent_vec = SENT + lane

    def ibody(i, carry):
        slab[pl.ds(i * LANES, LANES)] = sent_vec
        return carry

    lax.fori_loop(0, POS_PER_W // LANES, ibody, 0, unroll=8)

    for ci in range(K1_CHUNKS):
        base = ci * PCHUNK
        pltpu.sync_copy(b_hbm.at[pl.ds(base, PCHUNK)], bv)
        pltpu.sync_copy(y_hbm.at[pl.ds(base, PCHUNK)], yv)
        pltpu.sync_copy(x_hbm.at[pl.ds(base, PCHUNK)], xv)

        def gbody(g, carry):
            off = g * LANES
            bb = bv[pl.ds(off, LANES)]
            yy = yv[pl.ds(off, LANES)]
            xx = xv[pl.ds(off, LANES)]
            flat = bb * PLANE + xx * NY + yy
            loc = flat - lo
            mask = (loc >= 0) & (loc < POS_PER_W)
            safe = jnp.where(mask, loc, 0)
            pidv = (base + off) + lane
            plsc.store_scatter(slab, [safe], pidv, mask=mask)
            return carry

        lax.fori_loop(0, K1_GROUPS, gbody, 0, unroll=4)

    pltpu.sync_copy(slab, pid_hbm.at[pl.ds(lo, POS_PER_W)])


@functools.partial(
    pl.kernel,
    out_type=jax.ShapeDtypeStruct((B, C, NX, NY), jnp.float32),
    mesh=_mesh,
    compiler_params=_sc_params,
    scratch_types=[
        pltpu.VMEM((TW,), jnp.float32),
        pltpu.VMEM((TW,), jnp.float32),
        pltpu.VMEM((CP,), jnp.int32),
        pltpu.VMEM((CROWS, NY), jnp.float32),
        pltpu.VMEM((CROWS, NY), jnp.float32),
    ],
)
def _expand(pid_hbm, tab_hbm, out_hbm, t0, t1, pidb, ob0, ob1):
    wid = _wid()
    c0 = wid * 2
    pltpu.sync_copy(tab_hbm.at[c0], t0)
    pltpu.sync_copy(tab_hbm.at[c0 + 1], t1)

    for b in range(B):
        def cbody(k, carry):
            src = b * PLANE + k * CP
            pltpu.sync_copy(pid_hbm.at[pl.ds(src, CP)], pidb)

            def rbody(r, carry):
                rb = r * NY
                for q0, qn in ((0, 8), (8, 8), (16, 8), (24, 7)):
                    offs = [(q0 + j) * LANES for j in range(qn)]
                    idxs = [pidb[pl.ds(rb + o, LANES)] for o in offs]
                    v0s = [plsc.load_gather(t0, [ix]) for ix in idxs]
                    v1s = [plsc.load_gather(t1, [ix]) for ix in idxs]
                    for o, v0, v1 in zip(offs, v0s, v1s):
                        ob0[r, pl.ds(o, LANES)] = v0
                        ob1[r, pl.ds(o, LANES)] = v1
                return carry

            lax.fori_loop(0, CROWS, rbody, 0)

            pltpu.sync_copy(ob0, out_hbm.at[b, c0, pl.ds(k * CROWS, CROWS)])
            pltpu.sync_copy(ob1, out_hbm.at[b, c0 + 1, pl.ds(k * CROWS, CROWS)])
            return carry

        lax.fori_loop(0, K2_CHUNKS, cbody, 0)


def kernel(voxel_features, batch_idx, y_idx, x_idx):
    vf = voxel_features.astype(jnp.float32)
    bi = batch_idx.astype(jnp.int32)
    yi = y_idx.astype(jnp.int32)
    xi = x_idx.astype(jnp.int32)

    table = _transpose_table(vf)
    pid = _build_pid(bi, yi, xi)
    return jnp.swapaxes(_expand(pid, table), 2, 3)


# speedup vs baseline: 11.4657x; 1.2836x over previous
import functools

import jax
import jax.numpy as jnp
from jax import lax
from jax.experimental import pallas as pl
from jax.experimental.pallas import tpu as pltpu
from jax.experimental.pallas import tpu_sc as plsc

B, C, NY, NX = 4, 64, 496, 432
N_PILLARS = 48000
PLANE = NY * NX
TOT = B * PLANE
LANES = 16

NC, NS = 2, 16
NW = NC * NS
POS_PER_W = TOT // NW

TW = 48128
SENT = N_PILLARS

PCHUNK = 6000
K1_CHUNKS = N_PILLARS // PCHUNK
K1_GROUPS = PCHUNK // LANES

CROWS = 8
CP = CROWS * NY
K2_CHUNKS = NX // CROWS
K2_NCH = B * K2_CHUNKS
K2_GROUPS = NY // LANES

_mesh = plsc.VectorSubcoreMesh(core_axis_name="c", subcore_axis_name="s")
_sc_params = pltpu.CompilerParams(needs_layout_passes=False)


def _wid():
    return lax.axis_index("s") * NC + lax.axis_index("c")


def _tr_body(vf_ref, out_ref):
    i = pl.program_id(0)

    @pl.when(i < N_PILLARS // 128)
    def _():
        out_ref[...] = vf_ref[...].T

    @pl.when(i >= N_PILLARS // 128)
    def _():
        out_ref[...] = jnp.zeros((C, 128), jnp.float32)


def _transpose_table(vf):
    return pl.pallas_call(
        _tr_body,
        grid=(TW // 128,),
        in_specs=[pl.BlockSpec((128, C), lambda i: (jnp.minimum(i, N_PILLARS // 128 - 1), 0))],
        out_specs=pl.BlockSpec((C, 128), lambda i: (0, i)),
        out_shape=jax.ShapeDtypeStruct((C, TW), jnp.float32),
    )(vf)


@functools.partial(
    pl.kernel,
    out_type=jax.ShapeDtypeStruct((TOT,), jnp.int32),
    mesh=_mesh,
    compiler_params=_sc_params,
    scratch_types=[
        pltpu.VMEM((POS_PER_W,), jnp.int32),
        pltpu.VMEM((PCHUNK,), jnp.int32),
        pltpu.VMEM((PCHUNK,), jnp.int32),
        pltpu.VMEM((PCHUNK,), jnp.int32),
    ],
)
def _build_pid(b_hbm, y_hbm, x_hbm, pid_hbm, slab, bv, yv, xv):
    wid = _wid()
    lo = wid * POS_PER_W
    lane = lax.iota(jnp.int32, LANES)
    sent_vec = SENT + lane

    def ibody(i, carry):
        slab[pl.ds(i * LANES, LANES)] = sent_vec
        return carry

    lax.fori_loop(0, POS_PER_W // LANES, ibody, 0, unroll=8)

    for ci in range(K1_CHUNKS):
        base = ci * PCHUNK
        pltpu.sync_copy(b_hbm.at[pl.ds(base, PCHUNK)], bv)
        pltpu.sync_copy(y_hbm.at[pl.ds(base, PCHUNK)], yv)
        pltpu.sync_copy(x_hbm.at[pl.ds(base, PCHUNK)], xv)

        def gbody(g, carry):
            off = g * LANES
            bb = bv[pl.ds(off, LANES)]
            yy = yv[pl.ds(off, LANES)]
            xx = xv[pl.ds(off, LANES)]
            flat = bb * PLANE + xx * NY + yy
            loc = flat - lo
            mask = (loc >= 0) & (loc < POS_PER_W)
            safe = jnp.where(mask, loc, 0)
            pidv = (base + off) + lane
            plsc.store_scatter(slab, [safe], pidv, mask=mask)
            return carry

        lax.fori_loop(0, K1_GROUPS, gbody, 0, unroll=4)

    pltpu.sync_copy(slab, pid_hbm.at[pl.ds(lo, POS_PER_W)])


@functools.partial(
    pl.kernel,
    out_type=jax.ShapeDtypeStruct((B, C, NX, NY), jnp.float32),
    mesh=_mesh,
    compiler_params=_sc_params,
    scratch_types=[
        pltpu.VMEM((TW,), jnp.float32),
        pltpu.VMEM((TW,), jnp.float32),
        pltpu.VMEM((2, CP), jnp.int32),
        pltpu.VMEM((2, CROWS, NY), jnp.float32),
        pltpu.VMEM((2, CROWS, NY), jnp.float32),
        pltpu.SemaphoreType.DMA,
        pltpu.SemaphoreType.DMA,
        pltpu.SemaphoreType.DMA,
        pltpu.SemaphoreType.DMA,
    ],
)
def _expand(pid_hbm, tab_hbm, out_hbm, t0, t1, pidb, ob0, ob1, si0, si1, so0, so1):
    wid = _wid()
    c0 = wid * 2
    pltpu.sync_copy(tab_hbm.at[c0], t0)
    pltpu.sync_copy(tab_hbm.at[c0 + 1], t1)

    sin = (si0, si1)
    sout = (so0, so1)

    def bk(i):
        b = i // K2_CHUNKS
        return b, i - b * K2_CHUNKS

    def start_in(i, q):
        b, k = bk(i)
        pltpu.async_copy(pid_hbm.at[pl.ds(b * PLANE + k * CP, CP)], pidb.at[q], sin[q])

    def wait_in(q):
        pltpu.make_async_copy(pid_hbm.at[pl.ds(0, CP)], pidb.at[q], sin[q]).wait()

    def start_out(i, q):
        b, k = bk(i)
        pltpu.async_copy(ob0.at[q], out_hbm.at[b, c0, pl.ds(k * CROWS, CROWS)], sout[q])
        pltpu.async_copy(ob1.at[q], out_hbm.at[b, c0 + 1, pl.ds(k * CROWS, CROWS)], sout[q])

    def wait_out(q):
        pltpu.make_async_copy(ob0.at[q], out_hbm.at[0, 0, pl.ds(0, CROWS)], sout[q]).wait()
        pltpu.make_async_copy(ob1.at[q], out_hbm.at[0, 0, pl.ds(0, CROWS)], sout[q]).wait()

    def compute(q):
        def rbody(r, carry):
            rb = r * NY
            for q0, qn in ((0, 8), (8, 8), (16, 8), (24, 7)):
                offs = [(q0 + j) * LANES for j in range(qn)]
                idxs = [pidb[q, pl.ds(rb + o, LANES)] for o in offs]
                v0s = [plsc.load_gather(t0, [ix]) for ix in idxs]
                v1s = [plsc.load_gather(t1, [ix]) for ix in idxs]
                for o, v0, v1 in zip(offs, v0s, v1s):
                    ob0[q, r, pl.ds(o, LANES)] = v0
                    ob1[q, r, pl.ds(o, LANES)] = v1
            return carry

        lax.fori_loop(0, CROWS, rbody, 0)

    start_in(0, 0)
    start_in(1, 1)
    for q in (0, 1):
        wait_in(q)
        compute(q)
        start_out(q, q)
        start_in(q + 2, q)

    def pbody(j, carry):
        for q in (0, 1):
            i = 2 + 2 * j + q
            wait_in(q)
            wait_out(q)
            compute(q)
            start_out(i, q)
            start_in(jnp.minimum(i + 2, K2_NCH - 1), q)
        return carry

    lax.fori_loop(0, (K2_NCH - 2) // 2, pbody, 0)

    for q in (0, 1):
        wait_in(q)
        wait_out(q)


def kernel(voxel_features, batch_idx, y_idx, x_idx):
    vf = voxel_features.astype(jnp.float32)
    bi = batch_idx.astype(jnp.int32)
    yi = y_idx.astype(jnp.int32)
    xi = x_idx.astype(jnp.int32)

    table = _transpose_table(vf)
    pid = _build_pid(bi, yi, xi)
    return jnp.swapaxes(_expand(pid, table), 2, 3)
